# baseline (speedup 1.0000x reference)
import jax
import jax.numpy as jnp
from jax.experimental import pallas as pl
from jax.experimental.pallas import tpu as pltpu

NBLK = 2048
KP = 16


def _fused_body(spikes_ref, rates_ref, out_ref, cnt_ref):
    i = pl.program_id(0)

    r = rates_ref[...]
    s = spikes_ref[...]
    k = r.shape[1]

    norm = jnp.sum(jnp.abs(r), axis=1, keepdims=True)
    mx = jnp.max(r, axis=1, keepdims=True)
    lane = jax.lax.broadcasted_iota(jnp.int32, r.shape, 1)
    idx = jnp.min(jnp.where(r == mx, lane, k), axis=1, keepdims=True)
    w = mx / jnp.maximum(norm, 1e-12)

    lane16 = jax.lax.broadcasted_iota(jnp.int32, (r.shape[0], KP), 1)
    hit = lane16 == idx
    oh = jnp.where(hit, w, 0.0)
    ohc = jnp.where(hit, 1.0, 0.0)

    part = jax.lax.dot_general(
        s, oh, (((1,), (0,)), ((), ())),
        preferred_element_type=jnp.float32)
    cpart = jnp.sum(ohc, axis=0, keepdims=True)

    @pl.when(i == 0)
    def _():
        out_ref[...] = jnp.zeros_like(out_ref)
        cnt_ref[...] = jnp.zeros_like(cnt_ref)

    out_ref[...] += part
    cnt_ref[...] += cpart

    @pl.when(i == pl.num_programs(0) - 1)
    def _():
        cnt = cnt_ref[...]
        acc = out_ref[...]
        out_ref[...] = jnp.where(cnt > 0.0, acc / cnt, 0.0)


def kernel(spikes, rates):
    b, n = spikes.shape
    k = rates.shape[1]

    out = pl.pallas_call(
        _fused_body,
        grid=(n // NBLK,),
        in_specs=[
            pl.BlockSpec((b, NBLK), lambda i: (0, i)),
            pl.BlockSpec((NBLK, k), lambda i: (i, 0)),
        ],
        out_specs=pl.BlockSpec((b, KP), lambda i: (0, 0)),
        out_shape=jax.ShapeDtypeStruct((b, KP), jnp.float32),
        scratch_shapes=[pltpu.VMEM((1, KP), jnp.float32)],
        compiler_params=pltpu.CompilerParams(
            dimension_semantics=("arbitrary",),
        ),
    )(spikes, rates)
    return out[:, :k]
